# baseline (speedup 1.0000x reference)
import functools

import jax
import jax.numpy as jnp
from jax import lax
from jax.experimental import pallas as pl
from jax.experimental.pallas import tpu as pltpu
from jax.experimental.pallas import tpu_sc as plsc

B, NPG, D, L, R, NB, RDIM = 64, 64, 256, 3, 16, 4, 32
N = B * NPG
EPG = 1024
E = B * EPG

_SC_NUM_CORES = 2
_SC_NUM_SUBCORES = 16
NW = _SC_NUM_CORES * _SC_NUM_SUBCORES
GPW = B // NW
NPW = N // NW
EPW = E // NW
APW = GPW * NB * NPG * NPG

_PALLAS_CALL = pl.pallas_call


@functools.cache
def _make_sc_edge_kernel(l):
    return pl.kernel(
        functools.partial(_sc_edge_body, l),
        mesh=plsc.VectorSubcoreMesh(core_axis_name="c", subcore_axis_name="s"),
        compiler_params=pltpu.CompilerParams(needs_layout_passes=False),
        out_type=[
            jax.ShapeDtypeStruct((B * NB * NPG * NPG,), jnp.float32),
            jax.ShapeDtypeStruct((E,), jnp.float32),
            jax.ShapeDtypeStruct((N,), jnp.float32),
        ],
        scratch_types=[
            pltpu.VMEM((NPW,), jnp.float32),
            pltpu.VMEM((NPW,), jnp.float32),
            pltpu.VMEM((EPW,), jnp.int32),
            pltpu.VMEM((EPW,), jnp.int32),
            pltpu.VMEM((EPW,), jnp.int32),
            pltpu.VMEM((R * NB,), jnp.float32),
            pltpu.VMEM((APW,), jnp.float32),
            pltpu.VMEM((EPW,), jnp.float32),
            pltpu.VMEM((NPW,), jnp.float32),
            pltpu.SemaphoreType.DMA,
        ],
    )


def _sc_edge_body(l, s_hbm, d_hbm, src_hbm, dst_hbm, et_hbm, cf_hbm,
                  a_hbm, gates_hbm, deg_hbm,
                  s_v, d_v, src_v, dst_v, et_v, cf_v, a_v, g_v, deg_v, sem):
    wid = lax.axis_index("s") * _SC_NUM_CORES + lax.axis_index("c")
    ebase = wid * EPW
    nbase = wid * NPW
    copies = [
        pltpu.async_copy(s_hbm.at[pl.ds(nbase, NPW)], s_v, sem),
        pltpu.async_copy(d_hbm.at[pl.ds(nbase, NPW)], d_v, sem),
        pltpu.async_copy(src_hbm.at[pl.ds(ebase, EPW)], src_v, sem),
        pltpu.async_copy(dst_hbm.at[pl.ds(ebase, EPW)], dst_v, sem),
        pltpu.async_copy(et_hbm.at[pl.ds(ebase, EPW)], et_v, sem),
        pltpu.async_copy(cf_hbm.at[pl.ds(l * R * NB, R * NB)], cf_v, sem),
    ]

    zf = jnp.zeros((16,), jnp.float32)

    def zero_a(i, carry):
        a_v[pl.ds(i * 16, 16)] = zf
        return carry

    lax.fori_loop(0, APW // 16, zero_a, 0, unroll=32)

    if l == 0:
        def zero_deg(i, carry):
            deg_v[pl.ds(i * 16, 16)] = zf
            return carry

        lax.fori_loop(0, NPW // 16, zero_deg, 0, unroll=8)

    for c in copies:
        c.wait()

    of = jnp.ones((16,), jnp.float32)

    def edge_body(i, carry):
        s16 = src_v[pl.ds(i * 16, 16)]
        d16 = dst_v[pl.ds(i * 16, 16)]
        t16 = et_v[pl.ds(i * 16, 16)]
        sloc = jnp.bitwise_and(s16, NPW - 1)
        sval = plsc.load_gather(s_v, [sloc])
        dloc = jnp.bitwise_and(d16, NPW - 1)
        dval = plsc.load_gather(d_v, [dloc])
        gate = 1.0 / (1.0 + jnp.exp(-(sval + dval)))
        g_v[pl.ds(i * 16, 16)] = gate
        if l == 0:
            plsc.addupdate_scatter(deg_v, [dloc], of)
        base_idx = dloc * (NB * NPG) + jnp.bitwise_and(s16, NPG - 1)
        tb = t16 * NB
        for b in range(NB):
            cb = plsc.load_gather(cf_v, [tb + b])
            plsc.addupdate_scatter(a_v, [base_idx + b * NPG], gate * cb)
        return carry

    lax.fori_loop(0, EPW // 16, edge_body, 0, unroll=8)

    pltpu.sync_copy(a_v, a_hbm.at[pl.ds(wid * APW, APW)])
    pltpu.sync_copy(g_v, gates_hbm.at[pl.ds(ebase, EPW)])
    if l == 0:
        pltpu.sync_copy(deg_v, deg_hbm.at[pl.ds(nbase, NPW)])


_PREC = lax.Precision.DEFAULT


def _pre_body(x_ref, ws_ref, wd_ref, s_ref, d_ref):
    x = x_ref[...]
    cdims = (((1,), (1,)), ((), ()))
    s_ref[...] = lax.dot_general(x, ws_ref[0:1, :], cdims, precision=_PREC,
                                 preferred_element_type=jnp.float32)
    d_ref[...] = lax.dot_general(x, wd_ref[0:1, :], cdims, precision=_PREC,
                                 preferred_element_type=jnp.float32)


def _pre_gates(x, gws, gwd):
    return _PALLAS_CALL(
        _pre_body,
        out_shape=[jax.ShapeDtypeStruct((N, 1), jnp.float32),
                   jax.ShapeDtypeStruct((N, 1), jnp.float32)],
    )(x, gws, gwd)


GQ = 4
NQ = B // GQ
QN = GQ * NPG
QE = GQ * EPG


def _tc_layer_body(h_ref, a_ref, bases_ref, wself_ref, deg_ref, gates_ref,
                   ws_ref, wd_ref, hout_ref, s_ref, d_ref, kl_ref):
    h_q = h_ref[...]
    a_q = a_ref[0]
    hbs = [jnp.dot(h_q, bases_ref[0, b * D:(b + 1) * D, :], precision=_PREC,
                   preferred_element_type=jnp.float32) for b in range(NB)]
    accs = []
    for gi in range(GQ):
        hcat = jnp.concatenate(
            [hb[gi * NPG:(gi + 1) * NPG, :] for hb in hbs], axis=0)
        accs.append(jnp.dot(a_q[gi * NPG:(gi + 1) * NPG, :], hcat,
                            precision=_PREC,
                            preferred_element_type=jnp.float32))
    acc = jnp.concatenate(accs, axis=0)
    degc = jnp.maximum(deg_ref[0], 1.0)
    hn = acc / degc + jnp.dot(h_q, wself_ref[0], precision=_PREC,
                              preferred_element_type=jnp.float32)
    hn = jnp.maximum(hn, 0.0)
    hout_ref[...] = hn
    cdims = (((1,), (1,)), ((), ()))
    s_ref[...] = lax.dot_general(hn, ws_ref[0], cdims, precision=_PREC,
                                 preferred_element_type=jnp.float32)[None]
    d_ref[...] = lax.dot_general(hn, wd_ref[0], cdims, precision=_PREC,
                                 preferred_element_type=jnp.float32)[None]
    gg = gates_ref[0]
    kl = jnp.sum(gg * jnp.log(2.0 * gg + 1e-10)
                 + (1.0 - gg) * jnp.log(2.0 * (1.0 - gg) + 1e-10))
    kl_ref[...] = jnp.reshape(kl, (1, 1, 1))


def _tc_layer(l, h, a4, bases, wself, deg3, gates3, gws, gwd):
    nl = (l + 1) % L
    return _PALLAS_CALL(
        _tc_layer_body,
        grid=(NQ,),
        in_specs=[
            pl.BlockSpec((QN, D), lambda q: (q, 0)),
            pl.BlockSpec((1, QN, NB * NPG), lambda q: (q, 0, 0)),
            pl.BlockSpec((1, NB * D, D), lambda q, l=l: (l, 0, 0)),
            pl.BlockSpec((1, D, D), lambda q, l=l: (l, 0, 0)),
            pl.BlockSpec((1, QN, 1), lambda q: (q, 0, 0)),
            pl.BlockSpec((1, QE // 128, 128), lambda q: (q, 0, 0)),
            pl.BlockSpec((1, 1, D), lambda q, nl=nl: (nl, 0, 0)),
            pl.BlockSpec((1, 1, D), lambda q, nl=nl: (nl, 0, 0)),
        ],
        out_specs=[
            pl.BlockSpec((QN, D), lambda q: (q, 0)),
            pl.BlockSpec((1, QN, 1), lambda q: (q, 0, 0)),
            pl.BlockSpec((1, QN, 1), lambda q: (q, 0, 0)),
            pl.BlockSpec((1, 1, 1), lambda q: (q, 0, 0)),
        ],
        out_shape=[
            jax.ShapeDtypeStruct((N, D), jnp.float32),
            jax.ShapeDtypeStruct((NQ, QN, 1), jnp.float32),
            jax.ShapeDtypeStruct((NQ, QN, 1), jnp.float32),
            jax.ShapeDtypeStruct((NQ, 1, 1), jnp.float32),
        ],
    )(h, a4, bases, wself, deg3, gates3, gws, gwd)


def _epi_body(lab_ref, h1_ref, h2_ref, h3_ref, nid_ref, rel_ref, gsl_ref,
              gcn_ref, fcw_ref, fcb_ref,
              kl1_ref, kl2_ref, kl3_ref, out_ref, kl_out_ref):
    fcw = fcw_ref[...]
    f1 = fcw[:, 0:L * D]
    f2 = fcw[:, L * D:2 * L * D]
    f3 = fcw[:, 2 * L * D:3 * L * D]
    f4 = fcw[:, 3 * L * D:3 * L * D + RDIM]
    f5 = fcw[:, 3 * L * D + RDIM:3 * L * D + RDIM + D]
    q = pl.program_id(0)
    rc = jnp.concatenate([h1_ref[...], h2_ref[...], h3_ref[...]], axis=1)
    gidx = lax.broadcasted_iota(jnp.int32, (GQ, QN), 0)
    colg = lax.shift_right_logical(
        lax.broadcasted_iota(jnp.int32, (GQ, QN), 1), 6)
    ingraph = colg == gidx
    nidq = nid_ref[0]
    pool = jnp.where(ingraph, 1.0 / NPG, 0.0)
    hsel = jnp.where(ingraph & (nidq == 1), 1.0, 0.0)
    tsel = jnp.where(ingraph & (nidq == 2), 1.0, 0.0)
    selcat = jnp.concatenate([pool, hsel, tsel], axis=0)
    sels = jnp.dot(selcat, rc, precision=_PREC,
                   preferred_element_type=jnp.float32)
    gout = sels[0:GQ]
    head = sels[GQ:2 * GQ]
    tail = sels[2 * GQ:3 * GQ]
    h3q = h3_ref[...]
    z = jnp.maximum(jnp.dot(h3q, gsl_ref[...], precision=_PREC,
                            preferred_element_type=jnp.float32), 0.0)
    nrm = jnp.sqrt(jnp.sum(z * z, axis=1, keepdims=True)) + 1e-8
    zn = z / nrm
    cos = lax.dot_general(zn, zn, (((1,), (1,)), ((), ())), precision=_PREC,
                          preferred_element_type=jnp.float32)
    ii = lax.broadcasted_iota(jnp.int32, (QN, QN), 0)
    jj = lax.broadcasted_iota(jnp.int32, (QN, QN), 1)
    samegraph = lax.shift_right_logical(ii, 6) == lax.shift_right_logical(jj, 6)
    adj = (jnp.where(samegraph, jnp.maximum(cos, 0.0), 0.0)
           + (ii == jj).astype(jnp.float32))
    adjn = adj / jnp.sum(adj, axis=1, keepdims=True)
    gc = jnp.maximum(
        jnp.dot(adjn, jnp.dot(z, gcn_ref[...], precision=_PREC,
                              preferred_element_type=jnp.float32),
                precision=_PREC, preferred_element_type=jnp.float32), 0.0)
    grep = jnp.dot(pool, gc, precision=_PREC,
                   preferred_element_type=jnp.float32)
    selr = jnp.concatenate(
        [(lax.broadcasted_iota(jnp.int32, (1, R), 1)
          == lab_ref[q * GQ + gi]).astype(jnp.float32) for gi in range(GQ)],
        axis=0)
    rel = jnp.dot(selr, rel_ref[...], precision=_PREC,
                  preferred_element_type=jnp.float32)
    out4 = (jnp.sum(gout * f1, axis=1, keepdims=True)
            + jnp.sum(head * f2, axis=1, keepdims=True)
            + jnp.sum(tail * f3, axis=1, keepdims=True)
            + jnp.sum(rel * f4, axis=1, keepdims=True)
            + jnp.sum(grep * f5, axis=1, keepdims=True)
            + fcb_ref[0, 0])
    out_ref[...] = out4[None]

    @pl.when(q == 0)
    def _():
        kl = (jnp.sum(kl1_ref[...]) + jnp.sum(kl2_ref[...])
              + jnp.sum(kl3_ref[...])) / (3.0 * E)
        kl_out_ref[...] = jnp.reshape(kl, (1, 1, 1))


def _epilogue(rel_labels, h1, h2, h3, nidq, rel_emb, gsl_W, gcn_W,
              fcw, fcb, kl1, kl2, kl3):
    grid_spec = pltpu.PrefetchScalarGridSpec(
        num_scalar_prefetch=1,
        grid=(NQ,),
        in_specs=[
            pl.BlockSpec((QN, D), lambda q, lab: (q, 0)),
            pl.BlockSpec((QN, D), lambda q, lab: (q, 0)),
            pl.BlockSpec((QN, D), lambda q, lab: (q, 0)),
            pl.BlockSpec((1, 1, QN), lambda q, lab: (q, 0, 0)),
            pl.BlockSpec((R, RDIM), lambda q, lab: (0, 0)),
            pl.BlockSpec((D, D), lambda q, lab: (0, 0)),
            pl.BlockSpec((D, D), lambda q, lab: (0, 0)),
            pl.BlockSpec((1, 3 * L * D + RDIM + D), lambda q, lab: (0, 0)),
            pl.BlockSpec((1, 1), lambda q, lab: (0, 0)),
            pl.BlockSpec((NQ, 1, 1), lambda q, lab: (0, 0, 0)),
            pl.BlockSpec((NQ, 1, 1), lambda q, lab: (0, 0, 0)),
            pl.BlockSpec((NQ, 1, 1), lambda q, lab: (0, 0, 0)),
        ],
        out_specs=[
            pl.BlockSpec((1, GQ, 1), lambda q, lab: (q, 0, 0)),
            pl.BlockSpec((1, 1, 1), lambda q, lab: (0, 0, 0)),
        ],
    )
    return _PALLAS_CALL(
        _epi_body,
        grid_spec=grid_spec,
        out_shape=[jax.ShapeDtypeStruct((NQ, GQ, 1), jnp.float32),
                   jax.ShapeDtypeStruct((1, 1, 1), jnp.float32)],
    )(rel_labels, h1, h2, h3, nidq, rel_emb, gsl_W, gcn_W,
      fcw, fcb, kl1, kl2, kl3)


def _sc_layer(l, s, d, src, dst, et, cf):
    return _make_sc_edge_kernel(l)(s, d, src, dst, et, cf)


def kernel(x, edge_index, edge_type, node_id, node_graph_ids, rel_labels,
           rel_emb, rgcn_bases, rgcn_coeffs, rgcn_wself, gate_ws, gate_wd,
           gsl_W, gcn_W, fc_W, fc_b):
    src = edge_index[0].astype(jnp.int32)
    dst = edge_index[1].astype(jnp.int32)
    et = edge_type.astype(jnp.int32)
    cf = rgcn_coeffs.reshape(L * R * NB)
    bases = rgcn_bases.reshape(L, NB * D, D)
    s0, d0 = _pre_gates(x, gate_ws, gate_wd)
    s, d = s0.reshape(N), d0.reshape(N)
    h = x
    hs = []
    kls = []
    deg3 = None
    for l in range(L):
        a_flat, gates, deg = _sc_layer(l, s, d, src, dst, et, cf)
        if l == 0:
            deg3 = deg.reshape(NQ, QN, 1)
        a4 = a_flat.reshape(NQ, QN, NB * NPG)
        h, s3, d3, kl3 = _tc_layer(l, h, a4, bases, rgcn_wself, deg3,
                                   gates.reshape(NQ, QE // 128, 128),
                                   gate_ws.reshape(L, 1, D),
                                   gate_wd.reshape(L, 1, D))
        s, d = s3.reshape(N), d3.reshape(N)
        hs.append(h)
        kls.append(kl3)
    out3, klo = _epilogue(rel_labels.astype(jnp.int32), hs[0], hs[1], hs[2],
                          node_id.reshape(NQ, 1, QN).astype(jnp.int32),
                          rel_emb, gsl_W, gcn_W,
                          fc_W.reshape(1, 3 * L * D + RDIM + D),
                          fc_b.reshape(1, 1), kls[0], kls[1], kls[2])
    output = out3.reshape(B, 1)
    kl_loss = klo.reshape(())
    return (output, kl_loss)


# speedup vs baseline: 1.0542x; 1.0542x over previous
import functools

import jax
import jax.numpy as jnp
from jax import lax
from jax.experimental import pallas as pl
from jax.experimental.pallas import tpu as pltpu
from jax.experimental.pallas import tpu_sc as plsc

B, NPG, D, L, R, NB, RDIM = 64, 64, 256, 3, 16, 4, 32
N = B * NPG
EPG = 1024
E = B * EPG

_SC_NUM_CORES = 2
_SC_NUM_SUBCORES = 16
NW = _SC_NUM_CORES * _SC_NUM_SUBCORES
GPW = B // NW
NPW = N // NW
EPW = E // NW
APW = GPW * NB * NPG * NPG

_PALLAS_CALL = pl.pallas_call


@functools.cache
def _make_sc_edge_kernel(l):
    return pl.kernel(
        functools.partial(_sc_edge_body, l),
        mesh=plsc.VectorSubcoreMesh(core_axis_name="c", subcore_axis_name="s"),
        compiler_params=pltpu.CompilerParams(needs_layout_passes=False),
        out_type=[
            jax.ShapeDtypeStruct((B * NB * NPG * NPG,), jnp.float32),
            jax.ShapeDtypeStruct((E,), jnp.float32),
            jax.ShapeDtypeStruct((N,), jnp.float32),
        ],
        scratch_types=[
            pltpu.VMEM((NPW,), jnp.float32),
            pltpu.VMEM((NPW,), jnp.float32),
            pltpu.VMEM((EPW,), jnp.int32),
            pltpu.VMEM((EPW,), jnp.int32),
            pltpu.VMEM((EPW,), jnp.int32),
            pltpu.VMEM((R * NB,), jnp.float32),
            pltpu.VMEM((APW,), jnp.float32),
            pltpu.VMEM((EPW,), jnp.float32),
            pltpu.VMEM((NPW,), jnp.float32),
            pltpu.SemaphoreType.DMA,
        ],
    )


def _sc_edge_body(l, s_hbm, d_hbm, src_hbm, dst_hbm, et_hbm, cf_hbm,
                  a_hbm, gates_hbm, deg_hbm,
                  s_v, d_v, src_v, dst_v, et_v, cf_v, a_v, g_v, deg_v, sem):
    wid = lax.axis_index("s") * _SC_NUM_CORES + lax.axis_index("c")
    ebase = wid * EPW
    nbase = wid * NPW
    copies = [
        pltpu.async_copy(s_hbm.at[pl.ds(nbase, NPW)], s_v, sem),
        pltpu.async_copy(d_hbm.at[pl.ds(nbase, NPW)], d_v, sem),
        pltpu.async_copy(src_hbm.at[pl.ds(ebase, EPW)], src_v, sem),
        pltpu.async_copy(dst_hbm.at[pl.ds(ebase, EPW)], dst_v, sem),
        pltpu.async_copy(et_hbm.at[pl.ds(ebase, EPW)], et_v, sem),
        pltpu.async_copy(cf_hbm.at[pl.ds(l * R * NB, R * NB)], cf_v, sem),
    ]

    zf = jnp.zeros((16,), jnp.float32)

    def zero_a(i, carry):
        a_v[pl.ds(i * 16, 16)] = zf
        return carry

    lax.fori_loop(0, APW // 16, zero_a, 0, unroll=32)

    if l == 0:
        def zero_deg(i, carry):
            deg_v[pl.ds(i * 16, 16)] = zf
            return carry

        lax.fori_loop(0, NPW // 16, zero_deg, 0, unroll=8)

    for c in copies:
        c.wait()

    of = jnp.ones((16,), jnp.float32)

    def edge_body(i, carry):
        s16 = src_v[pl.ds(i * 16, 16)]
        d16 = dst_v[pl.ds(i * 16, 16)]
        t16 = et_v[pl.ds(i * 16, 16)]
        sloc = jnp.bitwise_and(s16, NPW - 1)
        sval = plsc.load_gather(s_v, [sloc])
        dloc = jnp.bitwise_and(d16, NPW - 1)
        dval = plsc.load_gather(d_v, [dloc])
        gate = 1.0 / (1.0 + jnp.exp(-(sval + dval)))
        g_v[pl.ds(i * 16, 16)] = gate
        if l == 0:
            plsc.addupdate_scatter(deg_v, [dloc], of)
        base_idx = dloc * (NB * NPG) + jnp.bitwise_and(s16, NPG - 1)
        tb = t16 * NB
        for b in range(NB):
            cb = plsc.load_gather(cf_v, [tb + b])
            plsc.addupdate_scatter(a_v, [base_idx + b * NPG], gate * cb)
        return carry

    lax.fori_loop(0, EPW // 16, edge_body, 0, unroll=8)

    pltpu.sync_copy(a_v, a_hbm.at[pl.ds(wid * APW, APW)])
    pltpu.sync_copy(g_v, gates_hbm.at[pl.ds(ebase, EPW)])
    if l == 0:
        pltpu.sync_copy(deg_v, deg_hbm.at[pl.ds(nbase, NPW)])


_PREC = lax.Precision.DEFAULT


def _pre_body(x_ref, ws_ref, wd_ref, s_ref, d_ref):
    x = x_ref[...]
    cdims = (((1,), (1,)), ((), ()))
    s_ref[...] = lax.dot_general(x, ws_ref[0:1, :], cdims, precision=_PREC,
                                 preferred_element_type=jnp.float32)
    d_ref[...] = lax.dot_general(x, wd_ref[0:1, :], cdims, precision=_PREC,
                                 preferred_element_type=jnp.float32)


def _pre_gates(x, gws, gwd):
    return _PALLAS_CALL(
        _pre_body,
        out_shape=[jax.ShapeDtypeStruct((N, 1), jnp.float32),
                   jax.ShapeDtypeStruct((N, 1), jnp.float32)],
    )(x, gws, gwd)


GQ = 4
NQ = B // GQ
QN = GQ * NPG
QE = GQ * EPG


def _tc_layer_body(h_ref, a_ref, bases_ref, wself_ref, deg_ref, gates_ref,
                   ws_ref, wd_ref, hout_ref, s_ref, d_ref, kl_ref):
    h_q = h_ref[...]
    a_q = a_ref[0]
    hbs = [jnp.dot(h_q, bases_ref[0, b * D:(b + 1) * D, :], precision=_PREC,
                   preferred_element_type=jnp.float32) for b in range(NB)]
    accs = []
    for gi in range(GQ):
        hcat = jnp.concatenate(
            [hb[gi * NPG:(gi + 1) * NPG, :] for hb in hbs], axis=0)
        accs.append(jnp.dot(a_q[gi * NPG:(gi + 1) * NPG, :], hcat,
                            precision=_PREC,
                            preferred_element_type=jnp.float32))
    acc = jnp.concatenate(accs, axis=0)
    degc = jnp.maximum(deg_ref[0], 1.0)
    hn = acc / degc + jnp.dot(h_q, wself_ref[0], precision=_PREC,
                              preferred_element_type=jnp.float32)
    hn = jnp.maximum(hn, 0.0)
    hout_ref[...] = hn
    cdims = (((1,), (1,)), ((), ()))
    s_ref[...] = lax.dot_general(hn, ws_ref[0], cdims, precision=_PREC,
                                 preferred_element_type=jnp.float32)[None]
    d_ref[...] = lax.dot_general(hn, wd_ref[0], cdims, precision=_PREC,
                                 preferred_element_type=jnp.float32)[None]
    gg = gates_ref[0]
    kl = jnp.sum(gg * jnp.log(2.0 * gg + 1e-10)
                 + (1.0 - gg) * jnp.log(2.0 * (1.0 - gg) + 1e-10))
    kl_ref[...] = jnp.reshape(kl, (1, 1, 1))


def _tc_layer(l, h, a4, bases, wself, deg3, gates3, gws, gwd):
    nl = (l + 1) % L
    return _PALLAS_CALL(
        _tc_layer_body,
        grid=(NQ,),
        in_specs=[
            pl.BlockSpec((QN, D), lambda q: (q, 0)),
            pl.BlockSpec((1, QN, NB * NPG), lambda q: (q, 0, 0)),
            pl.BlockSpec((1, NB * D, D), lambda q, l=l: (l, 0, 0)),
            pl.BlockSpec((1, D, D), lambda q, l=l: (l, 0, 0)),
            pl.BlockSpec((1, QN, 1), lambda q: (q, 0, 0)),
            pl.BlockSpec((1, QE // 128, 128), lambda q: (q, 0, 0)),
            pl.BlockSpec((1, 1, D), lambda q, nl=nl: (nl, 0, 0)),
            pl.BlockSpec((1, 1, D), lambda q, nl=nl: (nl, 0, 0)),
        ],
        out_specs=[
            pl.BlockSpec((QN, D), lambda q: (q, 0)),
            pl.BlockSpec((1, QN, 1), lambda q: (q, 0, 0)),
            pl.BlockSpec((1, QN, 1), lambda q: (q, 0, 0)),
            pl.BlockSpec((1, 1, 1), lambda q: (q, 0, 0)),
        ],
        out_shape=[
            jax.ShapeDtypeStruct((N, D), jnp.float32),
            jax.ShapeDtypeStruct((NQ, QN, 1), jnp.float32),
            jax.ShapeDtypeStruct((NQ, QN, 1), jnp.float32),
            jax.ShapeDtypeStruct((NQ, 1, 1), jnp.float32),
        ],
    )(h, a4, bases, wself, deg3, gates3, gws, gwd)


def _epi_body(lab_ref, h_ref, a_ref, bases_ref, wself_ref, deg_ref, gates_ref,
              h1_ref, nid_ref, rel_ref, gsl_ref,
              gcn_ref, fcw_ref, fcb_ref,
              kl1_ref, kl2_ref, out_ref, kl_out_ref):
    q = pl.program_id(0)
    h_q = h_ref[...]
    a_q = a_ref[0]
    hbs = [jnp.dot(h_q, bases_ref[0, b * D:(b + 1) * D, :], precision=_PREC,
                   preferred_element_type=jnp.float32) for b in range(NB)]
    accs = []
    for gi in range(GQ):
        hcat = jnp.concatenate(
            [hb[gi * NPG:(gi + 1) * NPG, :] for hb in hbs], axis=0)
        accs.append(jnp.dot(a_q[gi * NPG:(gi + 1) * NPG, :], hcat,
                            precision=_PREC,
                            preferred_element_type=jnp.float32))
    acc = jnp.concatenate(accs, axis=0)
    degc = jnp.maximum(deg_ref[0], 1.0)
    h3q = acc / degc + jnp.dot(h_q, wself_ref[0], precision=_PREC,
                               preferred_element_type=jnp.float32)
    h3q = jnp.maximum(h3q, 0.0)
    gg = gates_ref[0]
    klp = jnp.sum(gg * jnp.log(2.0 * gg + 1e-10)
                  + (1.0 - gg) * jnp.log(2.0 * (1.0 - gg) + 1e-10))

    @pl.when(q == 0)
    def _():
        kl_out_ref[...] = jnp.reshape(
            jnp.sum(kl1_ref[...]) + jnp.sum(kl2_ref[...]), (1, 1, 1))

    kl_out_ref[...] = kl_out_ref[...] + jnp.reshape(klp, (1, 1, 1))

    @pl.when(q == NQ - 1)
    def _():
        kl_out_ref[...] = kl_out_ref[...] / (3.0 * E)

    fcw = fcw_ref[...]
    f1 = fcw[:, 0:L * D]
    f2 = fcw[:, L * D:2 * L * D]
    f3 = fcw[:, 2 * L * D:3 * L * D]
    f4 = fcw[:, 3 * L * D:3 * L * D + RDIM]
    f5 = fcw[:, 3 * L * D + RDIM:3 * L * D + RDIM + D]
    rc = jnp.concatenate([h1_ref[...], h_q, h3q], axis=1)
    gidx = lax.broadcasted_iota(jnp.int32, (GQ, QN), 0)
    colg = lax.shift_right_logical(
        lax.broadcasted_iota(jnp.int32, (GQ, QN), 1), 6)
    ingraph = colg == gidx
    nidq = nid_ref[0]
    pool = jnp.where(ingraph, 1.0 / NPG, 0.0)
    hsel = jnp.where(ingraph & (nidq == 1), 1.0, 0.0)
    tsel = jnp.where(ingraph & (nidq == 2), 1.0, 0.0)
    selcat = jnp.concatenate([pool, hsel, tsel], axis=0)
    sels = jnp.dot(selcat, rc, precision=_PREC,
                   preferred_element_type=jnp.float32)
    gout = sels[0:GQ]
    head = sels[GQ:2 * GQ]
    tail = sels[2 * GQ:3 * GQ]
    z = jnp.maximum(jnp.dot(h3q, gsl_ref[...], precision=_PREC,
                            preferred_element_type=jnp.float32), 0.0)
    nrm = jnp.sqrt(jnp.sum(z * z, axis=1, keepdims=True)) + 1e-8
    zn = z / nrm
    cos = lax.dot_general(zn, zn, (((1,), (1,)), ((), ())), precision=_PREC,
                          preferred_element_type=jnp.float32)
    ii = lax.broadcasted_iota(jnp.int32, (QN, QN), 0)
    jj = lax.broadcasted_iota(jnp.int32, (QN, QN), 1)
    samegraph = lax.shift_right_logical(ii, 6) == lax.shift_right_logical(jj, 6)
    adj = (jnp.where(samegraph, jnp.maximum(cos, 0.0), 0.0)
           + (ii == jj).astype(jnp.float32))
    adjn = adj / jnp.sum(adj, axis=1, keepdims=True)
    gc = jnp.maximum(
        jnp.dot(adjn, jnp.dot(z, gcn_ref[...], precision=_PREC,
                              preferred_element_type=jnp.float32),
                precision=_PREC, preferred_element_type=jnp.float32), 0.0)
    grep = jnp.dot(pool, gc, precision=_PREC,
                   preferred_element_type=jnp.float32)
    selr = jnp.concatenate(
        [(lax.broadcasted_iota(jnp.int32, (1, R), 1)
          == lab_ref[q * GQ + gi]).astype(jnp.float32) for gi in range(GQ)],
        axis=0)
    rel = jnp.dot(selr, rel_ref[...], precision=_PREC,
                  preferred_element_type=jnp.float32)
    out4 = (jnp.sum(gout * f1, axis=1, keepdims=True)
            + jnp.sum(head * f2, axis=1, keepdims=True)
            + jnp.sum(tail * f3, axis=1, keepdims=True)
            + jnp.sum(rel * f4, axis=1, keepdims=True)
            + jnp.sum(grep * f5, axis=1, keepdims=True)
            + fcb_ref[0, 0])
    out_ref[...] = out4[None]


def _epilogue(rel_labels, h2, a4, bases, wself, deg3, gates3, h1, nidq,
              rel_emb, gsl_W, gcn_W, fcw, fcb, kl1, kl2):
    grid_spec = pltpu.PrefetchScalarGridSpec(
        num_scalar_prefetch=1,
        grid=(NQ,),
        in_specs=[
            pl.BlockSpec((QN, D), lambda q, lab: (q, 0)),
            pl.BlockSpec((1, QN, NB * NPG), lambda q, lab: (q, 0, 0)),
            pl.BlockSpec((1, NB * D, D), lambda q, lab: (L - 1, 0, 0)),
            pl.BlockSpec((1, D, D), lambda q, lab: (L - 1, 0, 0)),
            pl.BlockSpec((1, QN, 1), lambda q, lab: (q, 0, 0)),
            pl.BlockSpec((1, QE // 128, 128), lambda q, lab: (q, 0, 0)),
            pl.BlockSpec((QN, D), lambda q, lab: (q, 0)),
            pl.BlockSpec((1, 1, QN), lambda q, lab: (q, 0, 0)),
            pl.BlockSpec((R, RDIM), lambda q, lab: (0, 0)),
            pl.BlockSpec((D, D), lambda q, lab: (0, 0)),
            pl.BlockSpec((D, D), lambda q, lab: (0, 0)),
            pl.BlockSpec((1, 3 * L * D + RDIM + D), lambda q, lab: (0, 0)),
            pl.BlockSpec((1, 1), lambda q, lab: (0, 0)),
            pl.BlockSpec((NQ, 1, 1), lambda q, lab: (0, 0, 0)),
            pl.BlockSpec((NQ, 1, 1), lambda q, lab: (0, 0, 0)),
        ],
        out_specs=[
            pl.BlockSpec((1, GQ, 1), lambda q, lab: (q, 0, 0)),
            pl.BlockSpec((1, 1, 1), lambda q, lab: (0, 0, 0)),
        ],
    )
    return _PALLAS_CALL(
        _epi_body,
        grid_spec=grid_spec,
        out_shape=[jax.ShapeDtypeStruct((NQ, GQ, 1), jnp.float32),
                   jax.ShapeDtypeStruct((1, 1, 1), jnp.float32)],
    )(rel_labels, h2, a4, bases, wself, deg3, gates3, h1, nidq,
      rel_emb, gsl_W, gcn_W, fcw, fcb, kl1, kl2)


def _sc_layer(l, s, d, src, dst, et, cf):
    return _make_sc_edge_kernel(l)(s, d, src, dst, et, cf)


def kernel(x, edge_index, edge_type, node_id, node_graph_ids, rel_labels,
           rel_emb, rgcn_bases, rgcn_coeffs, rgcn_wself, gate_ws, gate_wd,
           gsl_W, gcn_W, fc_W, fc_b):
    src = edge_index[0].astype(jnp.int32)
    dst = edge_index[1].astype(jnp.int32)
    et = edge_type.astype(jnp.int32)
    cf = rgcn_coeffs.reshape(L * R * NB)
    bases = rgcn_bases.reshape(L, NB * D, D)
    s0, d0 = _pre_gates(x, gate_ws, gate_wd)
    s, d = s0.reshape(N), d0.reshape(N)
    h = x
    hs = []
    kls = []
    deg3 = None
    for l in range(L - 1):
        a_flat, gates, deg = _sc_layer(l, s, d, src, dst, et, cf)
        if l == 0:
            deg3 = deg.reshape(NQ, QN, 1)
        a4 = a_flat.reshape(NQ, QN, NB * NPG)
        h, s3, d3, kl3 = _tc_layer(l, h, a4, bases, rgcn_wself, deg3,
                                   gates.reshape(NQ, QE // 128, 128),
                                   gate_ws.reshape(L, 1, D),
                                   gate_wd.reshape(L, 1, D))
        s, d = s3.reshape(N), d3.reshape(N)
        hs.append(h)
        kls.append(kl3)
    a_flat, gates, _ = _sc_layer(L - 1, s, d, src, dst, et, cf)
    out3, klo = _epilogue(rel_labels.astype(jnp.int32), h,
                          a_flat.reshape(NQ, QN, NB * NPG), bases,
                          rgcn_wself, deg3,
                          gates.reshape(NQ, QE // 128, 128), hs[0],
                          node_id.reshape(NQ, 1, QN).astype(jnp.int32),
                          rel_emb, gsl_W, gcn_W,
                          fc_W.reshape(1, 3 * L * D + RDIM + D),
                          fc_b.reshape(1, 1), kls[0], kls[1])
    output = out3.reshape(B, 1)
    kl_loss = klo.reshape(())
    return (output, kl_loss)
